# baseline (speedup 1.0000x reference)
import functools

import jax
import jax.numpy as jnp
from jax import lax
from jax.experimental import pallas as pl
from jax.experimental.pallas import tpu as pltpu
from jax.experimental.pallas import tpu_sc as plsc

N = 10000
E = 160000
D = 128
H = 128

NW = 32
CH = 64
EPT = 5120
NCH = EPT // CH
EPAD = NW * EPT
NPAD = 10240
TRASH = N
RING = 4

BE = 2048
BN = 2000

_DOT = functools.partial(jnp.dot, preferred_element_type=jnp.float32,
                         precision=lax.Precision.HIGHEST)


def _ln(h, g, b, eps=1e-5):
    m = jnp.mean(h, axis=-1, keepdims=True)
    v = jnp.mean((h - m) ** 2, axis=-1, keepdims=True)
    return (h - m) / jnp.sqrt(v + eps) * g + b


_SC_MESH = plsc.VectorSubcoreMesh(core_axis_name="c", subcore_axis_name="s")


def _gather_body(xa_hbm, xb_hbm, row3_hbm, col3_hbm, g_hbm, idxr, idxc,
                 *rest):
    As = rest[0:RING]
    Bs = rest[RING:2 * RING]
    Os = rest[2 * RING:3 * RING]
    sgs = rest[3 * RING:4 * RING]
    sos = rest[4 * RING:5 * RING]
    w = lax.axis_index("c") * 16 + lax.axis_index("s")
    base = w * EPT
    pltpu.sync_copy(row3_hbm.at[w], idxr)
    pltpu.sync_copy(col3_hbm.at[w], idxc)

    def fire(c, p):
        pltpu.async_copy(xa_hbm.at[idxr.at[c]], As[p], sgs[p])
        pltpu.async_copy(xb_hbm.at[idxc.at[c]], Bs[p], sgs[p])

    for p in range(RING):
        fire(p, p)

    def outer(o, carry):
        for p in range(RING):
            c = o * RING + p
            A, B, O, sg, so = As[p], Bs[p], Os[p], sgs[p], sos[p]
            pltpu.make_async_copy(xa_hbm.at[idxr.at[c]], A, sg).wait()
            pltpu.make_async_copy(xb_hbm.at[idxc.at[c]], B, sg).wait()

            @pl.when(c >= RING)
            def _():
                pltpu.make_async_copy(
                    O, g_hbm.at[pl.ds(base + (c - RING) * CH, CH)],
                    so).wait()

            def ebody(e, carry2):
                for kk in range(8):
                    sl = pl.ds(kk * 16, 16)
                    O[e, sl] = A[e, sl] + B[e, sl]
                return carry2
            lax.fori_loop(0, CH, ebody, 0, unroll=2)
            pltpu.async_copy(O, g_hbm.at[pl.ds(base + c * CH, CH)], so)

            @pl.when(c + RING < NCH)
            def _():
                fire(c + RING, p)
        return carry
    lax.fori_loop(0, NCH // RING, outer, 0)
    for p in range(RING):
        c = NCH - RING + p
        pltpu.make_async_copy(
            Os[p], g_hbm.at[pl.ds(base + c * CH, CH)], sos[p]).wait()


@functools.partial(
    pl.kernel,
    out_type=jax.ShapeDtypeStruct((EPAD, D), jnp.float32),
    mesh=_SC_MESH,
    scratch_types=(
        [pltpu.VMEM((NCH, CH), jnp.int32)] * 2
        + [pltpu.VMEM((CH, D), jnp.float32)] * (3 * RING)
        + [pltpu.SemaphoreType.DMA] * (2 * RING)
    ),
)
def _gather_sc(xa, xb, row3, col3, g_out, *rest):
    _gather_body(xa, xb, row3, col3, g_out, *rest)


_KAHEAD = 2


def _scatter_body(ea_hbm, col3_hbm, out_hbm, idxc, acc, *rest):
    Es = rest[0:RING]
    sis = rest[RING:2 * RING]
    sas = rest[2 * RING:3 * RING]
    cid = lax.axis_index("c")
    sid = lax.axis_index("s")
    w = cid * 16 + sid
    base = w * EPT
    pltpu.sync_copy(col3_hbm.at[w], idxc)

    def zb(i, carry):
        for kk in range(8):
            Es[0][i, pl.ds(kk * 16, 16)] = jnp.zeros((16,), jnp.float32)
        return carry
    lax.fori_loop(0, CH, zb, 0)
    for j in range(NPAD // (16 * CH)):
        pltpu.sync_copy(Es[0], acc.at[pl.ds(sid * (NPAD // 16) + j * CH, CH)])
    plsc.subcore_barrier()

    def fire_in(c, p):
        pltpu.async_copy(ea_hbm.at[pl.ds(base + c * CH, CH)], Es[p], sis[p])

    for c in range(_KAHEAD):
        fire_in(c, c % RING)

    def outer(o, carry):
        for p in range(RING):
            c = o * RING + p
            Eb, si, sa = Es[p], sis[p], sas[p]
            pltpu.make_async_copy(
                ea_hbm.at[pl.ds(base + c * CH, CH)], Eb, si).wait()
            pltpu.async_copy(Eb, acc.at[idxc.at[c]], sa, add=True)

            @pl.when(c + _KAHEAD < NCH)
            def _():
                q = (p + _KAHEAD) % RING

                @pl.when(c + _KAHEAD >= RING)
                def _():
                    pltpu.make_async_copy(
                        Es[q],
                        acc.at[idxc.at[c + _KAHEAD - RING]], sas[q]).wait()
                fire_in(c + _KAHEAD, q)
        return carry
    lax.fori_loop(0, NCH // RING, outer, 0)
    for p in range(RING):
        c = NCH - RING + p
        pltpu.make_async_copy(Es[p], acc.at[idxc.at[c]], sas[p]).wait()
    plsc.subcore_barrier()
    r = sid * (NPAD // 16)
    pltpu.sync_copy(acc.at[pl.ds(r, NPAD // 16)],
                    out_hbm.at[cid, pl.ds(r, NPAD // 16)])


@functools.partial(
    pl.kernel,
    out_type=jax.ShapeDtypeStruct((2, NPAD, D), jnp.float32),
    mesh=_SC_MESH,
    scratch_types=(
        [pltpu.VMEM((NCH, CH), jnp.int32),
         pltpu.VMEM_SHARED((NPAD, D), jnp.float32)]
        + [pltpu.VMEM((CH, D), jnp.float32)] * RING
        + [pltpu.SemaphoreType.DMA] * (2 * RING)
    ),
)
def _scatter_sc(ea, col3, out, *rest):
    _scatter_body(ea, col3, out, *rest)


def _edge_mlp_body(g_ref, ea_ref, C_ref, W2_ref, b2_ref, W3_ref, b3_ref,
                   eg_ref, ebt_ref, out_ref):
    ea = ea_ref[...]
    h1 = jnp.maximum(g_ref[...] + _DOT(ea, C_ref[...]), 0.0)
    h2 = jnp.maximum(_DOT(h1, W2_ref[...]) + b2_ref[...], 0.0)
    e3 = _DOT(h2, W3_ref[...]) + b3_ref[...]
    out_ref[...] = _ln(e3, eg_ref[...], ebt_ref[...]) + ea


def _edge_stage(g, ea, C, W2, b2, W3, b3, eg, ebt):
    full = lambda s: pl.BlockSpec(s, lambda i: (0,) * len(s))
    return pl.pallas_call(
        _edge_mlp_body,
        grid=(EPAD // BE,),
        in_specs=[
            pl.BlockSpec((BE, H), lambda i: (i, 0)),
            pl.BlockSpec((BE, D), lambda i: (i, 0)),
            full((H, H)), full((H, H)), full((1, H)),
            full((H, D)), full((1, D)), full((1, D)), full((1, D)),
        ],
        out_specs=pl.BlockSpec((BE, D), lambda i: (i, 0)),
        out_shape=jax.ShapeDtypeStruct((EPAD, D), jnp.float32),
    )(g, ea, C, W2, b2.reshape(1, H), W3, b3.reshape(1, D),
      eg.reshape(1, D), ebt.reshape(1, D))


def _node_body(x_ref, a0_ref, a1_ref, W1a_ref, W1b_ref, b1_ref, W2_ref,
               b2_ref, W3_ref, b3_ref, ng_ref, nbt_ref, eA_ref, eB_ref,
               eb1_ref, x_out, xa_out, xb_out):
    x = x_ref[...]
    agg = a0_ref[...] + a1_ref[...]
    h1 = jnp.maximum(_DOT(x, W1a_ref[...]) + _DOT(agg, W1b_ref[...])
                     + b1_ref[...], 0.0)
    h2 = jnp.maximum(_DOT(h1, W2_ref[...]) + b2_ref[...], 0.0)
    x3 = _DOT(h2, W3_ref[...]) + b3_ref[...]
    xn = _ln(x3, ng_ref[...], nbt_ref[...]) + x
    x_out[...] = xn
    xa_out[...] = _DOT(xn, eA_ref[...]) + eb1_ref[...]
    xb_out[...] = _DOT(xn, eB_ref[...])


def _node_stage(x, a0, a1, W1a, W1b, b1, W2, b2, W3, b3, ng, nbt,
                eA, eB, eb1):
    full = lambda s: pl.BlockSpec(s, lambda i: (0,) * len(s))
    shp = jax.ShapeDtypeStruct((N, D), jnp.float32)
    return pl.pallas_call(
        _node_body,
        grid=(N // BN,),
        in_specs=[
            pl.BlockSpec((BN, D), lambda i: (i, 0)),
            pl.BlockSpec((BN, D), lambda i: (i, 0)),
            pl.BlockSpec((BN, D), lambda i: (i, 0)),
            full((D, H)), full((D, H)), full((1, H)),
            full((H, H)), full((1, H)), full((H, D)), full((1, D)),
            full((1, D)), full((1, D)),
            full((D, H)), full((D, H)), full((1, H)),
        ],
        out_specs=[pl.BlockSpec((BN, D), lambda i: (i, 0))] * 3,
        out_shape=[shp, shp, shp],
    )(x, a0, a1, W1a, W1b, b1.reshape(1, H), W2, b2.reshape(1, H),
      W3, b3.reshape(1, D), ng.reshape(1, D), nbt.reshape(1, D),
      eA, eB, eb1.reshape(1, H))


def _proj_body(x_ref, eA_ref, eB_ref, eb1_ref, xa_out, xb_out):
    x = x_ref[...]
    xa_out[...] = _DOT(x, eA_ref[...]) + eb1_ref[...]
    xb_out[...] = _DOT(x, eB_ref[...])


def _proj_stage(x, eA, eB, eb1):
    full = lambda s: pl.BlockSpec(s, lambda i: (0,) * len(s))
    shp = jax.ShapeDtypeStruct((N, H), jnp.float32)
    return pl.pallas_call(
        _proj_body,
        grid=(N // BN,),
        in_specs=[pl.BlockSpec((BN, D), lambda i: (i, 0)),
                  full((D, H)), full((D, H)), full((1, H))],
        out_specs=[pl.BlockSpec((BN, H), lambda i: (i, 0))] * 2,
        out_shape=[shp, shp],
    )(x, eA, eB, eb1.reshape(1, H))


def kernel(x, edge_attr, eW1, eb1, eW2, eb2, eW3, eb3, eg, ebt,
           nW1, nb1, nW2, nb2, nW3, nb3, ng, nbt, edge_index):
    T = eW1.shape[0]
    row = edge_index[0].astype(jnp.int32)
    col = edge_index[1].astype(jnp.int32)
    padi = jnp.zeros((EPAD - E,), jnp.int32)
    row3 = jnp.concatenate([row, padi]).reshape(NW, NCH, CH)
    colg3 = jnp.concatenate([col, padi]).reshape(NW, NCH, CH)
    cols3 = jnp.concatenate(
        [col, jnp.full((EPAD - E,), TRASH, jnp.int32)]).reshape(NW, NCH, CH)

    ea = jnp.concatenate(
        [edge_attr, jnp.zeros((EPAD - E, D), jnp.float32)], axis=0)
    xa, xb = _proj_stage(x, eW1[0, :D], eW1[0, D:2 * D], eb1[0])
    for i in range(T):
        g = _gather_sc(xa, xb, row3, colg3)
        ea = _edge_stage(g, ea, eW1[i, 2 * D:], eW2[i], eb2[i],
                         eW3[i], eb3[i], eg[i], ebt[i])
        parts = _scatter_sc(ea, cols3)
        j = min(i + 1, T - 1)
        x, xa, xb = _node_stage(
            x, parts[0], parts[1], nW1[i, :D], nW1[i, D:], nb1[i],
            nW2[i], nb2[i], nW3[i], nb3[i], ng[i], nbt[i],
            eW1[j, :D], eW1[j, D:2 * D], eb1[j])
    return (x, ea[:E])


# speedup vs baseline: 1.4774x; 1.4774x over previous
import functools

import jax
import jax.numpy as jnp
from jax import lax
from jax.experimental import pallas as pl
from jax.experimental.pallas import tpu as pltpu
from jax.experimental.pallas import tpu_sc as plsc

N = 10000
E = 160000
D = 128
H = 128

NW = 32
CH = 64
EPT = 5120
NCH = EPT // CH
EPAD = NW * EPT
NPAD = 10240
TRASH = N
RING = 4

BE = 2048
BN = 2000

_DOT = functools.partial(jnp.dot, preferred_element_type=jnp.float32,
                         precision=lax.Precision.DEFAULT)


def _ln(h, g, b, eps=1e-5):
    m = jnp.mean(h, axis=-1, keepdims=True)
    v = jnp.mean((h - m) ** 2, axis=-1, keepdims=True)
    return (h - m) / jnp.sqrt(v + eps) * g + b


_SC_MESH = plsc.VectorSubcoreMesh(core_axis_name="c", subcore_axis_name="s")


def _gather_body(xa_hbm, xb_hbm, row3_hbm, col3_hbm, g_hbm, idxr, idxc,
                 *rest):
    As = rest[0:RING]
    Bs = rest[RING:2 * RING]
    Os = rest[2 * RING:3 * RING]
    sgs = rest[3 * RING:4 * RING]
    sos = rest[4 * RING:5 * RING]
    w = lax.axis_index("c") * 16 + lax.axis_index("s")
    base = w * EPT
    pltpu.sync_copy(row3_hbm.at[w], idxr)
    pltpu.sync_copy(col3_hbm.at[w], idxc)

    def fire(c, p):
        pltpu.async_copy(xa_hbm.at[idxr.at[c]], As[p], sgs[p])
        pltpu.async_copy(xb_hbm.at[idxc.at[c]], Bs[p], sgs[p])

    for p in range(RING):
        fire(p, p)

    def outer(o, carry):
        for p in range(RING):
            c = o * RING + p
            A, B, O, sg, so = As[p], Bs[p], Os[p], sgs[p], sos[p]
            pltpu.make_async_copy(xa_hbm.at[idxr.at[c]], A, sg).wait()
            pltpu.make_async_copy(xb_hbm.at[idxc.at[c]], B, sg).wait()

            @pl.when(c >= RING)
            def _():
                pltpu.make_async_copy(
                    O, g_hbm.at[pl.ds(base + (c - RING) * CH, CH)],
                    so).wait()

            def ebody(e, carry2):
                for kk in range(8):
                    sl = pl.ds(kk * 16, 16)
                    O[e, sl] = A[e, sl] + B[e, sl]
                return carry2
            lax.fori_loop(0, CH, ebody, 0, unroll=2)
            pltpu.async_copy(O, g_hbm.at[pl.ds(base + c * CH, CH)], so)

            @pl.when(c + RING < NCH)
            def _():
                fire(c + RING, p)
        return carry
    lax.fori_loop(0, NCH // RING, outer, 0)
    for p in range(RING):
        c = NCH - RING + p
        pltpu.make_async_copy(
            Os[p], g_hbm.at[pl.ds(base + c * CH, CH)], sos[p]).wait()


@functools.partial(
    pl.kernel,
    out_type=jax.ShapeDtypeStruct((EPAD, D), jnp.float32),
    mesh=_SC_MESH,
    scratch_types=(
        [pltpu.VMEM((NCH, CH), jnp.int32)] * 2
        + [pltpu.VMEM((CH, D), jnp.float32)] * (3 * RING)
        + [pltpu.SemaphoreType.DMA] * (2 * RING)
    ),
)
def _gather_sc(xa, xb, row3, col3, g_out, *rest):
    _gather_body(xa, xb, row3, col3, g_out, *rest)


_KAHEAD = 2


def _scatter_body(ea_hbm, col3_hbm, out_hbm, idxc, acc, *rest):
    Es = rest[0:RING]
    sis = rest[RING:2 * RING]
    sas = rest[2 * RING:3 * RING]
    cid = lax.axis_index("c")
    sid = lax.axis_index("s")
    w = cid * 16 + sid
    base = w * EPT
    pltpu.sync_copy(col3_hbm.at[w], idxc)

    def zb(i, carry):
        for kk in range(8):
            Es[0][i, pl.ds(kk * 16, 16)] = jnp.zeros((16,), jnp.float32)
        return carry
    lax.fori_loop(0, CH, zb, 0)
    for j in range(NPAD // (16 * CH)):
        pltpu.sync_copy(Es[0], acc.at[pl.ds(sid * (NPAD // 16) + j * CH, CH)])
    plsc.subcore_barrier()

    def fire_in(c, p):
        pltpu.async_copy(ea_hbm.at[pl.ds(base + c * CH, CH)], Es[p], sis[p])

    for c in range(_KAHEAD):
        fire_in(c, c % RING)

    def outer(o, carry):
        for p in range(RING):
            c = o * RING + p
            Eb, si, sa = Es[p], sis[p], sas[p]
            pltpu.make_async_copy(
                ea_hbm.at[pl.ds(base + c * CH, CH)], Eb, si).wait()
            pltpu.async_copy(Eb, acc.at[idxc.at[c]], sa, add=True)

            @pl.when(c + _KAHEAD < NCH)
            def _():
                q = (p + _KAHEAD) % RING

                @pl.when(c + _KAHEAD >= RING)
                def _():
                    pltpu.make_async_copy(
                        Es[q],
                        acc.at[idxc.at[c + _KAHEAD - RING]], sas[q]).wait()
                fire_in(c + _KAHEAD, q)
        return carry
    lax.fori_loop(0, NCH // RING, outer, 0)
    for p in range(RING):
        c = NCH - RING + p
        pltpu.make_async_copy(Es[p], acc.at[idxc.at[c]], sas[p]).wait()
    plsc.subcore_barrier()
    r = sid * (NPAD // 16)
    pltpu.sync_copy(acc.at[pl.ds(r, NPAD // 16)],
                    out_hbm.at[cid, pl.ds(r, NPAD // 16)])


@functools.partial(
    pl.kernel,
    out_type=jax.ShapeDtypeStruct((2, NPAD, D), jnp.float32),
    mesh=_SC_MESH,
    scratch_types=(
        [pltpu.VMEM((NCH, CH), jnp.int32),
         pltpu.VMEM_SHARED((NPAD, D), jnp.float32)]
        + [pltpu.VMEM((CH, D), jnp.float32)] * RING
        + [pltpu.SemaphoreType.DMA] * (2 * RING)
    ),
)
def _scatter_sc(ea, col3, out, *rest):
    _scatter_body(ea, col3, out, *rest)


def _edge_mlp_body(g_ref, ea_ref, C_ref, W2_ref, b2_ref, W3_ref, b3_ref,
                   eg_ref, ebt_ref, out_ref):
    ea = ea_ref[...]
    h1 = jnp.maximum(g_ref[...] + _DOT(ea, C_ref[...]), 0.0)
    h2 = jnp.maximum(_DOT(h1, W2_ref[...]) + b2_ref[...], 0.0)
    e3 = _DOT(h2, W3_ref[...]) + b3_ref[...]
    out_ref[...] = _ln(e3, eg_ref[...], ebt_ref[...]) + ea


def _edge_stage(g, ea, C, W2, b2, W3, b3, eg, ebt):
    full = lambda s: pl.BlockSpec(s, lambda i: (0,) * len(s))
    return pl.pallas_call(
        _edge_mlp_body,
        grid=(EPAD // BE,),
        in_specs=[
            pl.BlockSpec((BE, H), lambda i: (i, 0)),
            pl.BlockSpec((BE, D), lambda i: (i, 0)),
            full((H, H)), full((H, H)), full((1, H)),
            full((H, D)), full((1, D)), full((1, D)), full((1, D)),
        ],
        out_specs=pl.BlockSpec((BE, D), lambda i: (i, 0)),
        out_shape=jax.ShapeDtypeStruct((EPAD, D), jnp.float32),
    )(g, ea, C, W2, b2.reshape(1, H), W3, b3.reshape(1, D),
      eg.reshape(1, D), ebt.reshape(1, D))


def _node_body(x_ref, a0_ref, a1_ref, W1a_ref, W1b_ref, b1_ref, W2_ref,
               b2_ref, W3_ref, b3_ref, ng_ref, nbt_ref, eA_ref, eB_ref,
               eb1_ref, x_out, xa_out, xb_out):
    x = x_ref[...]
    agg = a0_ref[...] + a1_ref[...]
    h1 = jnp.maximum(_DOT(x, W1a_ref[...]) + _DOT(agg, W1b_ref[...])
                     + b1_ref[...], 0.0)
    h2 = jnp.maximum(_DOT(h1, W2_ref[...]) + b2_ref[...], 0.0)
    x3 = _DOT(h2, W3_ref[...]) + b3_ref[...]
    xn = _ln(x3, ng_ref[...], nbt_ref[...]) + x
    x_out[...] = xn
    xa_out[...] = _DOT(xn, eA_ref[...]) + eb1_ref[...]
    xb_out[...] = _DOT(xn, eB_ref[...])


def _node_stage(x, a0, a1, W1a, W1b, b1, W2, b2, W3, b3, ng, nbt,
                eA, eB, eb1):
    full = lambda s: pl.BlockSpec(s, lambda i: (0,) * len(s))
    shp = jax.ShapeDtypeStruct((N, D), jnp.float32)
    return pl.pallas_call(
        _node_body,
        grid=(N // BN,),
        in_specs=[
            pl.BlockSpec((BN, D), lambda i: (i, 0)),
            pl.BlockSpec((BN, D), lambda i: (i, 0)),
            pl.BlockSpec((BN, D), lambda i: (i, 0)),
            full((D, H)), full((D, H)), full((1, H)),
            full((H, H)), full((1, H)), full((H, D)), full((1, D)),
            full((1, D)), full((1, D)),
            full((D, H)), full((D, H)), full((1, H)),
        ],
        out_specs=[pl.BlockSpec((BN, D), lambda i: (i, 0))] * 3,
        out_shape=[shp, shp, shp],
    )(x, a0, a1, W1a, W1b, b1.reshape(1, H), W2, b2.reshape(1, H),
      W3, b3.reshape(1, D), ng.reshape(1, D), nbt.reshape(1, D),
      eA, eB, eb1.reshape(1, H))


def _proj_body(x_ref, eA_ref, eB_ref, eb1_ref, xa_out, xb_out):
    x = x_ref[...]
    xa_out[...] = _DOT(x, eA_ref[...]) + eb1_ref[...]
    xb_out[...] = _DOT(x, eB_ref[...])


def _proj_stage(x, eA, eB, eb1):
    full = lambda s: pl.BlockSpec(s, lambda i: (0,) * len(s))
    shp = jax.ShapeDtypeStruct((N, H), jnp.float32)
    return pl.pallas_call(
        _proj_body,
        grid=(N // BN,),
        in_specs=[pl.BlockSpec((BN, D), lambda i: (i, 0)),
                  full((D, H)), full((D, H)), full((1, H))],
        out_specs=[pl.BlockSpec((BN, H), lambda i: (i, 0))] * 2,
        out_shape=[shp, shp],
    )(x, eA, eB, eb1.reshape(1, H))


def kernel(x, edge_attr, eW1, eb1, eW2, eb2, eW3, eb3, eg, ebt,
           nW1, nb1, nW2, nb2, nW3, nb3, ng, nbt, edge_index):
    T = eW1.shape[0]
    row = edge_index[0].astype(jnp.int32)
    col = edge_index[1].astype(jnp.int32)
    padi = jnp.zeros((EPAD - E,), jnp.int32)
    row3 = jnp.concatenate([row, padi]).reshape(NW, NCH, CH)
    colg3 = jnp.concatenate([col, padi]).reshape(NW, NCH, CH)
    cols3 = jnp.concatenate(
        [col, jnp.full((EPAD - E,), TRASH, jnp.int32)]).reshape(NW, NCH, CH)

    ea = jnp.concatenate(
        [edge_attr, jnp.zeros((EPAD - E, D), jnp.float32)], axis=0)
    xa, xb = _proj_stage(x, eW1[0, :D], eW1[0, D:2 * D], eb1[0])
    for i in range(T):
        g = _gather_sc(xa, xb, row3, colg3)
        ea = _edge_stage(g, ea, eW1[i, 2 * D:], eW2[i], eb2[i],
                         eW3[i], eb3[i], eg[i], ebt[i])
        parts = _scatter_sc(ea, cols3)
        j = min(i + 1, T - 1)
        x, xa, xb = _node_stage(
            x, parts[0], parts[1], nW1[i, :D], nW1[i, D:], nb1[i],
            nW2[i], nb2[i], nW3[i], nb3[i], ng[i], nbt[i],
            eW1[j, :D], eW1[j, D:2 * D], eb1[j])
    return (x, ea[:E])
